# baseline (speedup 1.0000x reference)
import functools
import math

import jax
import jax.numpy as jnp
from jax import lax
from jax.experimental import pallas as pl
from jax.experimental.pallas import tpu as pltpu
from jax.experimental.pallas import tpu_sc as plsc

N = 10000
E = 320000
G = 64
NC = 2
NS = 16
L = 16
W = NC * NS
B = 128
NCH = E // B
T = -(-NCH // W)
T2 = -(-(T + 1) // 2)
NP = 10240


def _zero_vec(ref, n16):
    for i in range(n16):
        ref[pl.ds(i * L, L)] = jnp.zeros((L,), jnp.float32)


@functools.lru_cache(maxsize=None)
def _edge_a(c):
    inv_sqrt_c = 1.0 / math.sqrt(float(c))
    mesh = plsc.VectorSubcoreMesh(core_axis_name="c", subcore_axis_name="s")

    @functools.partial(
        pl.kernel,
        out_type=[
            jax.ShapeDtypeStruct((E,), jnp.float32),
            jax.ShapeDtypeStruct((NC, NP), jnp.float32),
        ],
        mesh=mesh,
        scratch_types=[
            pltpu.VMEM((B,), jnp.int32), pltpu.VMEM((B,), jnp.int32),
            pltpu.VMEM((B,), jnp.int32), pltpu.VMEM((B,), jnp.int32),
            pltpu.VMEM((B, c), jnp.float32), pltpu.VMEM((B, c), jnp.float32),
            pltpu.VMEM((B, c), jnp.float32), pltpu.VMEM((B, c), jnp.float32),
            pltpu.VMEM((B,), jnp.float32), pltpu.VMEM((B,), jnp.float32),
            pltpu.VMEM((L, 2 * L), jnp.float32),
            pltpu.VMEM_SHARED((NP,), jnp.float32),
        ] + [pltpu.SemaphoreType.DMA] * 10,
        compiler_params=pltpu.CompilerParams(needs_layout_passes=False),
    )
    def kern(q_hbm, k_hbm, src_hbm, dst_hbm, ex_out, den_out,
             src0, src1, dst0, dst1, qr0, qr1, kr0, kr1, ex0, ex1,
             tmp, den_acc,
             si0, si1, sq0, sq1, sk0, sk1, se0, se1, sd0, sd1):
        sc = lax.axis_index("c")
        s = lax.axis_index("s")
        w = sc * NS + s
        srcs = (src0, src1)
        dsts = (dst0, dst1)
        qrs = (qr0, qr1)
        krs = (kr0, kr1)
        exs = (ex0, ex1)
        sis = (si0, si1)
        sqs = (sq0, sq1)
        sks = (sk0, sk1)
        ses = (se0, se1)
        sds = (sd0, sd1)
        lane = lax.iota(jnp.int32, L)

        _zero_vec(ex0, B // L)

        def _zd(i, _):
            pltpu.sync_copy(ex0, den_acc.at[pl.ds(640 * s + B * i, B)])
            return 0
        lax.fori_loop(0, 5, _zd, 0)
        plsc.subcore_barrier()

        def issue_idx(j, b):
            e = (j * W + w) * B
            pltpu.async_copy(src_hbm.at[pl.ds(e, B)], srcs[b], sis[b])
            pltpu.async_copy(dst_hbm.at[pl.ds(e, B)], dsts[b], sis[b])

        def wait_idx(b):
            pltpu.make_async_copy(src_hbm.at[pl.ds(0, B)], srcs[b],
                                  sis[b]).wait()
            pltpu.make_async_copy(dst_hbm.at[pl.ds(0, B)], dsts[b],
                                  sis[b]).wait()

        H = B // 2

        def issue_gathers(b):
            pltpu.async_copy(q_hbm.at[dsts[b].at[pl.ds(0, H)]],
                             qrs[b].at[pl.ds(0, H)], sqs[b])
            pltpu.async_copy(q_hbm.at[dsts[b].at[pl.ds(H, H)]],
                             qrs[b].at[pl.ds(H, H)], sqs[b])
            pltpu.async_copy(k_hbm.at[srcs[b].at[pl.ds(0, H)]],
                             krs[b].at[pl.ds(0, H)], sks[b])
            pltpu.async_copy(k_hbm.at[srcs[b].at[pl.ds(H, H)]],
                             krs[b].at[pl.ds(H, H)], sks[b])

        def wait_gathers(b):
            for o in (0, H):
                pltpu.make_async_copy(q_hbm.at[dsts[b].at[pl.ds(o, H)]],
                                      qrs[b].at[pl.ds(o, H)], sqs[b]).wait()
                pltpu.make_async_copy(k_hbm.at[srcs[b].at[pl.ds(o, H)]],
                                      krs[b].at[pl.ds(o, H)], sks[b]).wait()

        def dot(b):
            qr, kr, exv = qrs[b], krs[b], exs[b]

            def group(g, _):
                ex_acc = jnp.zeros((L,), jnp.float32)
                for i in range(L):
                    e = g * L + i
                    a0 = qr[e, pl.ds(0, L)] * kr[e, pl.ds(0, L)]
                    a1 = qr[e, pl.ds(L, L)] * kr[e, pl.ds(L, L)]
                    for j in range(2, c // L, 2):
                        a0 = a0 + qr[e, pl.ds(j * L, L)] * kr[e, pl.ds(j * L, L)]
                        a1 = a1 + (qr[e, pl.ds((j + 1) * L, L)]
                                   * kr[e, pl.ds((j + 1) * L, L)])
                    r = a0 + a1
                    for sh in (8, 4, 2, 1):
                        tmp[i, pl.ds(0, L)] = r
                        r = r + tmp[i, pl.ds(sh, L)]
                    ex_acc = jnp.where(lane == i, r[0], ex_acc)
                exv[pl.ds(g * L, L)] = jnp.exp(ex_acc * inv_sqrt_c)
                return 0
            lax.fori_loop(0, B // L, group, 0)

        def issue_out(j, b):
            e = (j * W + w) * B
            pltpu.async_copy(exs[b], ex_out.at[pl.ds(e, B)], ses[b])
            pltpu.async_copy(exs[b], den_acc.at[dsts[b]], sds[b], add=True)

        def wait_out(b):
            pltpu.make_async_copy(exs[b], ex_out.at[pl.ds(0, B)],
                                  ses[b]).wait()
            pltpu.make_async_copy(exs[b], den_acc.at[dsts[b]],
                                  sds[b]).wait()

        issue_idx(0, 0)
        wait_idx(0)
        issue_gathers(0)

        def outer(jj, _):
            for b in (0, 1):
                j = jj * 2 + b
                nb = 1 - b
                t = j * W + w
                ex_j = t < NCH
                ex_jn = (t + W) < NCH

                @pl.when(ex_j)
                def _():
                    wait_gathers(b)

                @pl.when(ex_jn & (j >= 1))
                def _():
                    wait_out(nb)

                @pl.when(ex_jn)
                def _():
                    issue_idx(j + 1, nb)

                @pl.when(ex_j)
                def _():
                    dot(b)
                    issue_out(j, b)

                @pl.when(ex_jn)
                def _():
                    wait_idx(nb)
                    issue_gathers(nb)
            return 0
        lax.fori_loop(0, T2, outer, 0)

        jlast = (NCH - 1 - w) // W
        for b in (0, 1):
            @pl.when((jlast >= 0) & (lax.rem(jlast, 2) == b))
            def _():
                wait_out(b)

            @pl.when((jlast >= 1) & (lax.rem(jlast - 1, 2) == b))
            def _():
                wait_out(b)

        plsc.subcore_barrier()

        @pl.when(s < 8)
        def _export_den():
            pltpu.sync_copy(den_acc.at[pl.ds(1280 * s, 1280)],
                            den_out.at[sc].at[pl.ds(1280 * s, 1280)])

    return kern


@functools.lru_cache(maxsize=None)
def _edge_b(c):
    mesh = plsc.VectorSubcoreMesh(core_axis_name="c", subcore_axis_name="s")

    @functools.partial(
        pl.kernel,
        out_type=[jax.ShapeDtypeStruct((NC, N, c), jnp.float32)],
        mesh=mesh,
        scratch_types=[
            pltpu.VMEM((B,), jnp.int32), pltpu.VMEM((B,), jnp.int32),
            pltpu.VMEM((B,), jnp.int32), pltpu.VMEM((B,), jnp.int32),
            pltpu.VMEM((B, c), jnp.float32), pltpu.VMEM((B, c), jnp.float32),
            pltpu.VMEM((B,), jnp.float32), pltpu.VMEM((B,), jnp.float32),
            pltpu.VMEM_SHARED((N, c), jnp.float32),
        ] + [pltpu.SemaphoreType.DMA] * 8,
        compiler_params=pltpu.CompilerParams(needs_layout_passes=False),
    )
    def kern(v_hbm, src_hbm, dst_hbm, ex_hbm, num_out,
             src0, src1, dst0, dst1, vr0, vr1, ex0, ex1,
             num_acc,
             si0, si1, sv0, sv1, sx0, sx1, sn0, sn1):
        sc = lax.axis_index("c")
        s = lax.axis_index("s")
        w = sc * NS + s
        srcs = (src0, src1)
        dsts = (dst0, dst1)
        vrs = (vr0, vr1)
        exs = (ex0, ex1)
        sis = (si0, si1)
        svs = (sv0, sv1)
        sxs = (sx0, sx1)
        sns = (sn0, sn1)
        lane = lax.iota(jnp.int32, L)

        def _z1(i, _):
            r = i // (c // L)
            col = lax.rem(i, c // L) * L
            vr0[r, pl.ds(col, L)] = jnp.zeros((L,), jnp.float32)
            return 0
        lax.fori_loop(0, B * c // L, _z1, 0)

        def _zn(i, _):
            pltpu.sync_copy(vr0.at[pl.ds(0, 125)],
                            num_acc.at[pl.ds(625 * s + 125 * i, 125)])
            return 0
        lax.fori_loop(0, 5, _zn, 0)
        plsc.subcore_barrier()

        def issue_idx(j, b):
            e = (j * W + w) * B
            pltpu.async_copy(src_hbm.at[pl.ds(e, B)], srcs[b], sis[b])
            pltpu.async_copy(dst_hbm.at[pl.ds(e, B)], dsts[b], sis[b])

        def wait_idx(b):
            pltpu.make_async_copy(src_hbm.at[pl.ds(0, B)], srcs[b],
                                  sis[b]).wait()
            pltpu.make_async_copy(dst_hbm.at[pl.ds(0, B)], dsts[b],
                                  sis[b]).wait()

        H = B // 2

        def issue_gathers(j, b):
            e = (j * W + w) * B
            pltpu.async_copy(v_hbm.at[srcs[b].at[pl.ds(0, H)]],
                             vrs[b].at[pl.ds(0, H)], svs[b])
            pltpu.async_copy(v_hbm.at[srcs[b].at[pl.ds(H, H)]],
                             vrs[b].at[pl.ds(H, H)], svs[b])
            pltpu.async_copy(ex_hbm.at[pl.ds(e, B)], exs[b], sxs[b])

        def wait_gathers(b):
            for o in (0, H):
                pltpu.make_async_copy(v_hbm.at[srcs[b].at[pl.ds(o, H)]],
                                      vrs[b].at[pl.ds(o, H)], svs[b]).wait()
            pltpu.make_async_copy(ex_hbm.at[pl.ds(0, B)], exs[b],
                                  sxs[b]).wait()

        def scale(b):
            vr, exv = vrs[b], exs[b]

            def group(g, _):
                ex = exv[pl.ds(g * L, L)]
                for i in range(L):
                    e = g * L + i
                    sc_e = ex[i]
                    for j in range(c // L):
                        vr[e, pl.ds(j * L, L)] = vr[e, pl.ds(j * L, L)] * sc_e
                return 0
            lax.fori_loop(0, B // L, group, 0)

        def issue_out(b):
            pltpu.async_copy(vrs[b], num_acc.at[dsts[b]], sns[b], add=True)

        def wait_out(b):
            pltpu.make_async_copy(vrs[b], num_acc.at[dsts[b]],
                                  sns[b]).wait()

        issue_idx(0, 0)
        wait_idx(0)
        issue_gathers(0, 0)

        def outer(jj, _):
            for b in (0, 1):
                j = jj * 2 + b
                nb = 1 - b
                t = j * W + w
                ex_j = t < NCH
                ex_jn = (t + W) < NCH

                @pl.when(ex_j)
                def _():
                    wait_gathers(b)

                @pl.when(ex_jn & (j >= 1))
                def _():
                    wait_out(nb)

                @pl.when(ex_jn)
                def _():
                    issue_idx(j + 1, nb)

                @pl.when(ex_j)
                def _():
                    scale(b)
                    issue_out(b)

                @pl.when(ex_jn)
                def _():
                    wait_idx(nb)
                    issue_gathers(j + 1, nb)
            return 0
        lax.fori_loop(0, T2, outer, 0)

        jlast = (NCH - 1 - w) // W
        for b in (0, 1):
            @pl.when((jlast >= 0) & (lax.rem(jlast, 2) == b))
            def _():
                wait_out(b)

            @pl.when((jlast >= 1) & (lax.rem(jlast - 1, 2) == b))
            def _():
                wait_out(b)

        plsc.subcore_barrier()

        @pl.when(s < 10)
        def _export_num():
            pltpu.sync_copy(num_acc.at[pl.ds(1000 * s, 1000)],
                            num_out.at[sc].at[pl.ds(1000 * s, 1000)])

    return kern


_RB = 2000


def _proj(h, wt, bc, c):
    din = h.shape[1]

    def body(h_ref, w_ref, b_ref, q_ref, k_ref, v_ref, s_ref):
        o = jnp.dot(h_ref[...], w_ref[...],
                    preferred_element_type=jnp.float32,
                    precision=lax.Precision.HIGHEST) + b_ref[...]
        q_ref[...] = o[:, 0:c]
        k_ref[...] = o[:, c:2 * c]
        v_ref[...] = o[:, 2 * c:3 * c]
        s_ref[...] = o[:, 3 * c:4 * c]

    return pl.pallas_call(
        body,
        grid=(N // _RB,),
        in_specs=[
            pl.BlockSpec((_RB, din), lambda i: (i, 0)),
            pl.BlockSpec((din, 4 * c), lambda i: (0, 0)),
            pl.BlockSpec((1, 4 * c), lambda i: (0, 0)),
        ],
        out_specs=[pl.BlockSpec((_RB, c), lambda i: (i, 0))] * 4,
        out_shape=[jax.ShapeDtypeStruct((N, c), jnp.float32)] * 4,
    )(h, wt, bc)


def _post_proj(num0, num1, den0, den1, xr, u, w, g, b, wt, bc, c, cq, xw):
    tw = 3 * cq + xw

    def body(n0, n1, d0, d1, xr_ref, u_ref, w_ref, g_ref, b_ref,
             wt_ref, bc_ref, q_ref, k_ref, v_ref, s_ref):
        num = n0[...] + n1[...]
        den = d0[...] + d1[...] + 1e-16
        out = num / den
        xrb = xr_ref[...]
        bet = jax.nn.sigmoid(
            jnp.dot(out, u_ref[...].T, preferred_element_type=jnp.float32,
                    precision=lax.Precision.HIGHEST)
            + jnp.dot(xrb, w_ref[...].T, preferred_element_type=jnp.float32,
                      precision=lax.Precision.HIGHEST))
        h = bet * xrb + (1.0 - bet) * out
        mu = jnp.mean(h, axis=1, keepdims=True)
        var = jnp.mean((h - mu) ** 2, axis=1, keepdims=True)
        hn = (h - mu) / jnp.sqrt(var + 1e-5) * g_ref[...] + b_ref[...]
        hn = jnp.maximum(hn, 0.0)
        o = jnp.dot(hn, wt_ref[...], preferred_element_type=jnp.float32,
                    precision=lax.Precision.HIGHEST) + bc_ref[...]
        q_ref[...] = o[:, 0:cq]
        k_ref[...] = o[:, cq:2 * cq]
        v_ref[...] = o[:, 2 * cq:3 * cq]
        s_ref[...] = o[:, 3 * cq:3 * cq + xw]

    return pl.pallas_call(
        body,
        grid=(N // _RB,),
        in_specs=[
            pl.BlockSpec((_RB, c), lambda i: (i, 0)),
            pl.BlockSpec((_RB, c), lambda i: (i, 0)),
            pl.BlockSpec((_RB, 1), lambda i: (i, 0)),
            pl.BlockSpec((_RB, 1), lambda i: (i, 0)),
            pl.BlockSpec((_RB, c), lambda i: (i, 0)),
            pl.BlockSpec((1, c), lambda i: (0, 0)),
            pl.BlockSpec((1, c), lambda i: (0, 0)),
            pl.BlockSpec((1, c), lambda i: (0, 0)),
            pl.BlockSpec((1, c), lambda i: (0, 0)),
            pl.BlockSpec((c, tw), lambda i: (0, 0)),
            pl.BlockSpec((1, tw), lambda i: (0, 0)),
        ],
        out_specs=([pl.BlockSpec((_RB, cq), lambda i: (i, 0))] * 3
                   + [pl.BlockSpec((_RB, xw), lambda i: (i, 0))]),
        out_shape=([jax.ShapeDtypeStruct((N, cq), jnp.float32)] * 3
                   + [jax.ShapeDtypeStruct((N, xw), jnp.float32)]),
    )(num0, num1, den0, den1, xr, u, w, g, b, wt, bc)


def _final(num0, num1, den0, den1, xr, u, w, bi, wmu_t, bmu, wlv_t, blv, eps):
    c = 64

    def body(n0, n1, d0, d1, xr_ref, u_ref, w_ref, bi_ref,
             wmu_ref, bmu_ref, wlv_ref, blv_ref, eps_ref,
             z_ref, zmu_ref, zlv_ref):
        num = n0[...] + n1[...]
        den = d0[...] + d1[...] + 1e-16
        out = num / den
        xrb = xr_ref[...]
        bet = jax.nn.sigmoid(
            jnp.dot(out, u_ref[...].T, preferred_element_type=jnp.float32,
                    precision=lax.Precision.HIGHEST)
            + jnp.dot(xrb, w_ref[...].T, preferred_element_type=jnp.float32,
                      precision=lax.Precision.HIGHEST))
        h = bet * xrb + (1.0 - bet) * out
        oh = (bi_ref[...] == lax.broadcasted_iota(jnp.int32, (N, G), 1)
              ).astype(jnp.float32)
        sums = lax.dot_general(oh, h, (((0,), (0,)), ((), ())),
                               preferred_element_type=jnp.float32,
                               precision=lax.Precision.HIGHEST)
        cnt = jnp.sum(oh, axis=0)[:, None]
        pooled = sums / jnp.maximum(cnt, 1.0)
        zmu = jnp.dot(pooled, wmu_ref[...],
                      preferred_element_type=jnp.float32,
                      precision=lax.Precision.HIGHEST) + bmu_ref[...]
        zlv = jnp.dot(pooled, wlv_ref[...],
                      preferred_element_type=jnp.float32,
                      precision=lax.Precision.HIGHEST) + blv_ref[...]
        z_ref[...] = eps_ref[...] * jnp.exp(0.5 * zlv) + zmu
        zmu_ref[...] = zmu
        zlv_ref[...] = zlv

    return pl.pallas_call(
        body,
        grid=(1,),
        in_specs=[
            pl.BlockSpec((N, c), lambda i: (0, 0)),
            pl.BlockSpec((N, c), lambda i: (0, 0)),
            pl.BlockSpec((N, 1), lambda i: (0, 0)),
            pl.BlockSpec((N, 1), lambda i: (0, 0)),
            pl.BlockSpec((N, c), lambda i: (0, 0)),
            pl.BlockSpec((1, c), lambda i: (0, 0)),
            pl.BlockSpec((1, c), lambda i: (0, 0)),
            pl.BlockSpec((N, 1), lambda i: (0, 0)),
            pl.BlockSpec((c, 128), lambda i: (0, 0)),
            pl.BlockSpec((1, 128), lambda i: (0, 0)),
            pl.BlockSpec((c, 128), lambda i: (0, 0)),
            pl.BlockSpec((1, 128), lambda i: (0, 0)),
            pl.BlockSpec((G, 128), lambda i: (0, 0)),
        ],
        out_specs=[pl.BlockSpec((G, 128), lambda i: (0, 0))] * 3,
        out_shape=[jax.ShapeDtypeStruct((G, 128), jnp.float32)] * 3,
    )(num0, num1, den0, den1, xr, u, w, bi, wmu_t, bmu, wlv_t, blv, eps)


def _cat_w(p, pad_to=None):
    ws = [p['Wq'].T, p['Wk'].T, p['Wv'].T]
    if pad_to is not None:
        ws = [jnp.pad(wi, ((0, 0), (0, pad_to - wi.shape[1]))) for wi in ws]
    ws.append(p['Ws'].T)
    return jnp.concatenate(ws, axis=1)


def _cat_b(p, pad_to=None):
    bs = [p['bq'], p['bk'], p['bv']]
    if pad_to is not None:
        bs = [jnp.pad(bi, (0, pad_to - bi.shape[0])) for bi in bs]
    bs.append(p['bs'])
    return jnp.concatenate(bs)[None, :]


def _gate_uw(p, c):
    wb = p['Wb'][0]
    return ((wb[:c] + wb[2 * c:])[None, :],
            (wb[c:2 * c] - wb[2 * c:])[None, :])


def _edge(q, k, v, src, dst):
    exh, den = _edge_a(128)(q, k, src, dst)
    num, = _edge_b(128)(v, src, dst, exh)
    return num, den


def kernel(x, edge_attr, edge_index, batch_index, params):
    del edge_attr
    src = edge_index[0]
    dst = edge_index[1]
    convs = params['convs']
    norms = params['norms']

    q, k, v, xr = _proj(x, _cat_w(convs[0]), _cat_b(convs[0]), 128)
    for i in range(3):
        pad = 128 if i == 2 else None
        xw = 64 if i == 2 else 128
        num, den = _edge(q, k, v, src, dst)
        u, w = _gate_uw(convs[i], 128)
        q, k, v, xr = _post_proj(
            num[0], num[1], den[0][:N, None], den[1][:N, None], xr, u, w,
            norms[i]['g'][None, :], norms[i]['b'][None, :],
            _cat_w(convs[i + 1], pad), _cat_b(convs[i + 1], pad),
            128, 128, xw)

    num, den = _edge(q, k, v, src, dst)
    u3, w3 = _gate_uw(convs[3], 64)
    eps = jax.random.normal(jax.random.key(42), (G, 128), jnp.float32)
    return _final(num[0][:, :64], num[1][:, :64],
                  den[0][:N, None], den[1][:N, None], xr,
                  u3, w3, batch_index[:, None].astype(jnp.int32),
                  params['Wmu'].T, params['bmu'][None, :],
                  params['Wlv'].T, params['blv'][None, :], eps)


# speedup vs baseline: 1.3957x; 1.3957x over previous
import functools
import math

import jax
import jax.numpy as jnp
from jax import lax
from jax.experimental import pallas as pl
from jax.experimental.pallas import tpu as pltpu
from jax.experimental.pallas import tpu_sc as plsc

N = 10000
E = 320000
G = 64
NC = 2
NS = 16
L = 16
W = NC * NS
B = 128
NCH = E // B
T = -(-NCH // W)
T2 = -(-(T + 1) // 2)
NP = 10240


def _zero_vec(ref, n16):
    for i in range(n16):
        ref[pl.ds(i * L, L)] = jnp.zeros((L,), jnp.float32)


@functools.lru_cache(maxsize=None)
def _edge_a(c):
    inv_sqrt_c = 1.0 / math.sqrt(float(c))
    mesh = plsc.VectorSubcoreMesh(core_axis_name="c", subcore_axis_name="s")

    @functools.partial(
        pl.kernel,
        out_type=[
            jax.ShapeDtypeStruct((E,), jnp.float32),
            jax.ShapeDtypeStruct((NC, NP), jnp.float32),
        ],
        mesh=mesh,
        scratch_types=[
            pltpu.VMEM((B,), jnp.int32), pltpu.VMEM((B,), jnp.int32),
            pltpu.VMEM((B,), jnp.int32), pltpu.VMEM((B,), jnp.int32),
            pltpu.VMEM((B, c), jnp.float32), pltpu.VMEM((B, c), jnp.float32),
            pltpu.VMEM((B, c), jnp.float32), pltpu.VMEM((B, c), jnp.float32),
            pltpu.VMEM((B,), jnp.float32), pltpu.VMEM((B,), jnp.float32),
            pltpu.VMEM_SHARED((NP,), jnp.float32),
        ] + [pltpu.SemaphoreType.DMA] * 10,
        compiler_params=pltpu.CompilerParams(needs_layout_passes=False),
    )
    def kern(q_hbm, k_hbm, src_hbm, dst_hbm, ex_out, den_out,
             src0, src1, dst0, dst1, qr0, qr1, kr0, kr1, ex0, ex1,
             den_acc,
             si0, si1, sq0, sq1, sk0, sk1, se0, se1, sd0, sd1):
        sc = lax.axis_index("c")
        s = lax.axis_index("s")
        w = sc * NS + s
        srcs = (src0, src1)
        dsts = (dst0, dst1)
        qrs = (qr0, qr1)
        krs = (kr0, kr1)
        exs = (ex0, ex1)
        sis = (si0, si1)
        sqs = (sq0, sq1)
        sks = (sk0, sk1)
        ses = (se0, se1)
        sds = (sd0, sd1)
        lane = lax.iota(jnp.int32, L)

        _zero_vec(ex0, B // L)

        def _zd(i, _):
            pltpu.sync_copy(ex0, den_acc.at[pl.ds(640 * s + B * i, B)])
            return 0
        lax.fori_loop(0, 5, _zd, 0)
        plsc.subcore_barrier()

        def issue_idx(j, b):
            e = (j * W + w) * B
            pltpu.async_copy(src_hbm.at[pl.ds(e, B)], srcs[b], sis[b])
            pltpu.async_copy(dst_hbm.at[pl.ds(e, B)], dsts[b], sis[b])

        def wait_idx(b):
            pltpu.make_async_copy(src_hbm.at[pl.ds(0, B)], srcs[b],
                                  sis[b]).wait()
            pltpu.make_async_copy(dst_hbm.at[pl.ds(0, B)], dsts[b],
                                  sis[b]).wait()

        H = B // 2

        def issue_gathers(b):
            pltpu.async_copy(q_hbm.at[dsts[b].at[pl.ds(0, H)]],
                             qrs[b].at[pl.ds(0, H)], sqs[b])
            pltpu.async_copy(q_hbm.at[dsts[b].at[pl.ds(H, H)]],
                             qrs[b].at[pl.ds(H, H)], sqs[b])
            pltpu.async_copy(k_hbm.at[srcs[b].at[pl.ds(0, H)]],
                             krs[b].at[pl.ds(0, H)], sks[b])
            pltpu.async_copy(k_hbm.at[srcs[b].at[pl.ds(H, H)]],
                             krs[b].at[pl.ds(H, H)], sks[b])

        def wait_gathers(b):
            for o in (0, H):
                pltpu.make_async_copy(q_hbm.at[dsts[b].at[pl.ds(o, H)]],
                                      qrs[b].at[pl.ds(o, H)], sqs[b]).wait()
                pltpu.make_async_copy(k_hbm.at[srcs[b].at[pl.ds(o, H)]],
                                      krs[b].at[pl.ds(o, H)], sks[b]).wait()

        FB = 32

        def dot(b):
            qr, kr, exv = qrs[b], krs[b], exs[b]

            def group(g, _):
                rows = lane + g * L

                def fblk(fb, accs):
                    accs = list(accs)
                    for i in range(FB):
                        colf = jnp.bitwise_and(lane + (fb * FB + i), c - 1)
                        p = (plsc.load_gather(qr, [rows, colf])
                             * plsc.load_gather(kr, [rows, colf]))
                        accs[i & 3] = accs[i & 3] + p
                    return tuple(accs)
                z = jnp.zeros((L,), jnp.float32)
                accs = lax.fori_loop(0, c // FB, fblk, (z, z, z, z))
                acc = (accs[0] + accs[1]) + (accs[2] + accs[3])
                exv[pl.ds(g * L, L)] = jnp.exp(acc * inv_sqrt_c)
                return 0
            lax.fori_loop(0, B // L, group, 0)

        def issue_out(j, b):
            e = (j * W + w) * B
            pltpu.async_copy(exs[b], ex_out.at[pl.ds(e, B)], ses[b])
            pltpu.async_copy(exs[b], den_acc.at[dsts[b]], sds[b], add=True)

        def wait_out(b):
            pltpu.make_async_copy(exs[b], ex_out.at[pl.ds(0, B)],
                                  ses[b]).wait()
            pltpu.make_async_copy(exs[b], den_acc.at[dsts[b]],
                                  sds[b]).wait()

        issue_idx(0, 0)
        wait_idx(0)
        issue_gathers(0)

        def outer(jj, _):
            for b in (0, 1):
                j = jj * 2 + b
                nb = 1 - b
                t = j * W + w
                ex_j = t < NCH
                ex_jn = (t + W) < NCH

                @pl.when(ex_j)
                def _():
                    wait_gathers(b)

                @pl.when(ex_jn & (j >= 1))
                def _():
                    wait_out(nb)

                @pl.when(ex_jn)
                def _():
                    issue_idx(j + 1, nb)

                @pl.when(ex_j)
                def _():
                    dot(b)
                    issue_out(j, b)

                @pl.when(ex_jn)
                def _():
                    wait_idx(nb)
                    issue_gathers(nb)
            return 0
        lax.fori_loop(0, T2, outer, 0)

        jlast = (NCH - 1 - w) // W
        for b in (0, 1):
            @pl.when((jlast >= 0) & (lax.rem(jlast, 2) == b))
            def _():
                wait_out(b)

            @pl.when((jlast >= 1) & (lax.rem(jlast - 1, 2) == b))
            def _():
                wait_out(b)

        plsc.subcore_barrier()

        @pl.when(s < 8)
        def _export_den():
            pltpu.sync_copy(den_acc.at[pl.ds(1280 * s, 1280)],
                            den_out.at[sc].at[pl.ds(1280 * s, 1280)])

    return kern


@functools.lru_cache(maxsize=None)
def _edge_b(c):
    mesh = plsc.VectorSubcoreMesh(core_axis_name="c", subcore_axis_name="s")

    @functools.partial(
        pl.kernel,
        out_type=[jax.ShapeDtypeStruct((NC, N, c), jnp.float32)],
        mesh=mesh,
        scratch_types=[
            pltpu.VMEM((B,), jnp.int32), pltpu.VMEM((B,), jnp.int32),
            pltpu.VMEM((B,), jnp.int32), pltpu.VMEM((B,), jnp.int32),
            pltpu.VMEM((B, c), jnp.float32), pltpu.VMEM((B, c), jnp.float32),
            pltpu.VMEM((B,), jnp.float32), pltpu.VMEM((B,), jnp.float32),
            pltpu.VMEM_SHARED((N, c), jnp.float32),
        ] + [pltpu.SemaphoreType.DMA] * 8,
        compiler_params=pltpu.CompilerParams(needs_layout_passes=False),
    )
    def kern(v_hbm, src_hbm, dst_hbm, ex_hbm, num_out,
             src0, src1, dst0, dst1, vr0, vr1, ex0, ex1,
             num_acc,
             si0, si1, sv0, sv1, sx0, sx1, sn0, sn1):
        sc = lax.axis_index("c")
        s = lax.axis_index("s")
        w = sc * NS + s
        srcs = (src0, src1)
        dsts = (dst0, dst1)
        vrs = (vr0, vr1)
        exs = (ex0, ex1)
        sis = (si0, si1)
        svs = (sv0, sv1)
        sxs = (sx0, sx1)
        sns = (sn0, sn1)
        lane = lax.iota(jnp.int32, L)

        def _z1(i, _):
            r = i // (c // L)
            col = lax.rem(i, c // L) * L
            vr0[r, pl.ds(col, L)] = jnp.zeros((L,), jnp.float32)
            return 0
        lax.fori_loop(0, B * c // L, _z1, 0)

        def _zn(i, _):
            pltpu.sync_copy(vr0.at[pl.ds(0, 125)],
                            num_acc.at[pl.ds(625 * s + 125 * i, 125)])
            return 0
        lax.fori_loop(0, 5, _zn, 0)
        plsc.subcore_barrier()

        def issue_idx(j, b):
            e = (j * W + w) * B
            pltpu.async_copy(src_hbm.at[pl.ds(e, B)], srcs[b], sis[b])
            pltpu.async_copy(dst_hbm.at[pl.ds(e, B)], dsts[b], sis[b])

        def wait_idx(b):
            pltpu.make_async_copy(src_hbm.at[pl.ds(0, B)], srcs[b],
                                  sis[b]).wait()
            pltpu.make_async_copy(dst_hbm.at[pl.ds(0, B)], dsts[b],
                                  sis[b]).wait()

        H = B // 2

        def issue_gathers(j, b):
            e = (j * W + w) * B
            pltpu.async_copy(v_hbm.at[srcs[b].at[pl.ds(0, H)]],
                             vrs[b].at[pl.ds(0, H)], svs[b])
            pltpu.async_copy(v_hbm.at[srcs[b].at[pl.ds(H, H)]],
                             vrs[b].at[pl.ds(H, H)], svs[b])
            pltpu.async_copy(ex_hbm.at[pl.ds(e, B)], exs[b], sxs[b])

        def wait_gathers(b):
            for o in (0, H):
                pltpu.make_async_copy(v_hbm.at[srcs[b].at[pl.ds(o, H)]],
                                      vrs[b].at[pl.ds(o, H)], svs[b]).wait()
            pltpu.make_async_copy(ex_hbm.at[pl.ds(0, B)], exs[b],
                                  sxs[b]).wait()

        def scale(b):
            vr, exv = vrs[b], exs[b]

            def group(g, _):
                ex = exv[pl.ds(g * L, L)]
                for i in range(L):
                    e = g * L + i
                    sc_e = ex[i]
                    for j in range(c // L):
                        vr[e, pl.ds(j * L, L)] = vr[e, pl.ds(j * L, L)] * sc_e
                return 0
            lax.fori_loop(0, B // L, group, 0)

        def issue_out(b):
            pltpu.async_copy(vrs[b], num_acc.at[dsts[b]], sns[b], add=True)

        def wait_out(b):
            pltpu.make_async_copy(vrs[b], num_acc.at[dsts[b]],
                                  sns[b]).wait()

        issue_idx(0, 0)
        wait_idx(0)
        issue_gathers(0, 0)

        def outer(jj, _):
            for b in (0, 1):
                j = jj * 2 + b
                nb = 1 - b
                t = j * W + w
                ex_j = t < NCH
                ex_jn = (t + W) < NCH

                @pl.when(ex_j)
                def _():
                    wait_gathers(b)

                @pl.when(ex_jn & (j >= 1))
                def _():
                    wait_out(nb)

                @pl.when(ex_jn)
                def _():
                    issue_idx(j + 1, nb)

                @pl.when(ex_j)
                def _():
                    scale(b)
                    issue_out(b)

                @pl.when(ex_jn)
                def _():
                    wait_idx(nb)
                    issue_gathers(j + 1, nb)
            return 0
        lax.fori_loop(0, T2, outer, 0)

        jlast = (NCH - 1 - w) // W
        for b in (0, 1):
            @pl.when((jlast >= 0) & (lax.rem(jlast, 2) == b))
            def _():
                wait_out(b)

            @pl.when((jlast >= 1) & (lax.rem(jlast - 1, 2) == b))
            def _():
                wait_out(b)

        plsc.subcore_barrier()

        @pl.when(s < 10)
        def _export_num():
            pltpu.sync_copy(num_acc.at[pl.ds(1000 * s, 1000)],
                            num_out.at[sc].at[pl.ds(1000 * s, 1000)])

    return kern


_RB = 2000


def _proj(h, wt, bc, c):
    din = h.shape[1]

    def body(h_ref, w_ref, b_ref, q_ref, k_ref, v_ref, s_ref):
        o = jnp.dot(h_ref[...], w_ref[...],
                    preferred_element_type=jnp.float32,
                    precision=lax.Precision.HIGHEST) + b_ref[...]
        q_ref[...] = o[:, 0:c]
        k_ref[...] = o[:, c:2 * c]
        v_ref[...] = o[:, 2 * c:3 * c]
        s_ref[...] = o[:, 3 * c:4 * c]

    return pl.pallas_call(
        body,
        grid=(N // _RB,),
        in_specs=[
            pl.BlockSpec((_RB, din), lambda i: (i, 0)),
            pl.BlockSpec((din, 4 * c), lambda i: (0, 0)),
            pl.BlockSpec((1, 4 * c), lambda i: (0, 0)),
        ],
        out_specs=[pl.BlockSpec((_RB, c), lambda i: (i, 0))] * 4,
        out_shape=[jax.ShapeDtypeStruct((N, c), jnp.float32)] * 4,
    )(h, wt, bc)


def _post_proj(num0, num1, den0, den1, xr, u, w, g, b, wt, bc, c, cq, xw):
    tw = 3 * cq + xw

    def body(n0, n1, d0, d1, xr_ref, u_ref, w_ref, g_ref, b_ref,
             wt_ref, bc_ref, q_ref, k_ref, v_ref, s_ref):
        num = n0[...] + n1[...]
        den = d0[...] + d1[...] + 1e-16
        out = num / den
        xrb = xr_ref[...]
        bet = jax.nn.sigmoid(
            jnp.dot(out, u_ref[...].T, preferred_element_type=jnp.float32,
                    precision=lax.Precision.HIGHEST)
            + jnp.dot(xrb, w_ref[...].T, preferred_element_type=jnp.float32,
                      precision=lax.Precision.HIGHEST))
        h = bet * xrb + (1.0 - bet) * out
        mu = jnp.mean(h, axis=1, keepdims=True)
        var = jnp.mean((h - mu) ** 2, axis=1, keepdims=True)
        hn = (h - mu) / jnp.sqrt(var + 1e-5) * g_ref[...] + b_ref[...]
        hn = jnp.maximum(hn, 0.0)
        o = jnp.dot(hn, wt_ref[...], preferred_element_type=jnp.float32,
                    precision=lax.Precision.HIGHEST) + bc_ref[...]
        q_ref[...] = o[:, 0:cq]
        k_ref[...] = o[:, cq:2 * cq]
        v_ref[...] = o[:, 2 * cq:3 * cq]
        s_ref[...] = o[:, 3 * cq:3 * cq + xw]

    return pl.pallas_call(
        body,
        grid=(N // _RB,),
        in_specs=[
            pl.BlockSpec((_RB, c), lambda i: (i, 0)),
            pl.BlockSpec((_RB, c), lambda i: (i, 0)),
            pl.BlockSpec((_RB, 1), lambda i: (i, 0)),
            pl.BlockSpec((_RB, 1), lambda i: (i, 0)),
            pl.BlockSpec((_RB, c), lambda i: (i, 0)),
            pl.BlockSpec((1, c), lambda i: (0, 0)),
            pl.BlockSpec((1, c), lambda i: (0, 0)),
            pl.BlockSpec((1, c), lambda i: (0, 0)),
            pl.BlockSpec((1, c), lambda i: (0, 0)),
            pl.BlockSpec((c, tw), lambda i: (0, 0)),
            pl.BlockSpec((1, tw), lambda i: (0, 0)),
        ],
        out_specs=([pl.BlockSpec((_RB, cq), lambda i: (i, 0))] * 3
                   + [pl.BlockSpec((_RB, xw), lambda i: (i, 0))]),
        out_shape=([jax.ShapeDtypeStruct((N, cq), jnp.float32)] * 3
                   + [jax.ShapeDtypeStruct((N, xw), jnp.float32)]),
    )(num0, num1, den0, den1, xr, u, w, g, b, wt, bc)


def _final(num0, num1, den0, den1, xr, u, w, bi, wmu_t, bmu, wlv_t, blv, eps):
    c = 64

    def body(n0, n1, d0, d1, xr_ref, u_ref, w_ref, bi_ref,
             wmu_ref, bmu_ref, wlv_ref, blv_ref, eps_ref,
             z_ref, zmu_ref, zlv_ref):
        num = n0[...] + n1[...]
        den = d0[...] + d1[...] + 1e-16
        out = num / den
        xrb = xr_ref[...]
        bet = jax.nn.sigmoid(
            jnp.dot(out, u_ref[...].T, preferred_element_type=jnp.float32,
                    precision=lax.Precision.HIGHEST)
            + jnp.dot(xrb, w_ref[...].T, preferred_element_type=jnp.float32,
                      precision=lax.Precision.HIGHEST))
        h = bet * xrb + (1.0 - bet) * out
        oh = (bi_ref[...] == lax.broadcasted_iota(jnp.int32, (N, G), 1)
              ).astype(jnp.float32)
        sums = lax.dot_general(oh, h, (((0,), (0,)), ((), ())),
                               preferred_element_type=jnp.float32,
                               precision=lax.Precision.HIGHEST)
        cnt = jnp.sum(oh, axis=0)[:, None]
        pooled = sums / jnp.maximum(cnt, 1.0)
        zmu = jnp.dot(pooled, wmu_ref[...],
                      preferred_element_type=jnp.float32,
                      precision=lax.Precision.HIGHEST) + bmu_ref[...]
        zlv = jnp.dot(pooled, wlv_ref[...],
                      preferred_element_type=jnp.float32,
                      precision=lax.Precision.HIGHEST) + blv_ref[...]
        z_ref[...] = eps_ref[...] * jnp.exp(0.5 * zlv) + zmu
        zmu_ref[...] = zmu
        zlv_ref[...] = zlv

    return pl.pallas_call(
        body,
        grid=(1,),
        in_specs=[
            pl.BlockSpec((N, c), lambda i: (0, 0)),
            pl.BlockSpec((N, c), lambda i: (0, 0)),
            pl.BlockSpec((N, 1), lambda i: (0, 0)),
            pl.BlockSpec((N, 1), lambda i: (0, 0)),
            pl.BlockSpec((N, c), lambda i: (0, 0)),
            pl.BlockSpec((1, c), lambda i: (0, 0)),
            pl.BlockSpec((1, c), lambda i: (0, 0)),
            pl.BlockSpec((N, 1), lambda i: (0, 0)),
            pl.BlockSpec((c, 128), lambda i: (0, 0)),
            pl.BlockSpec((1, 128), lambda i: (0, 0)),
            pl.BlockSpec((c, 128), lambda i: (0, 0)),
            pl.BlockSpec((1, 128), lambda i: (0, 0)),
            pl.BlockSpec((G, 128), lambda i: (0, 0)),
        ],
        out_specs=[pl.BlockSpec((G, 128), lambda i: (0, 0))] * 3,
        out_shape=[jax.ShapeDtypeStruct((G, 128), jnp.float32)] * 3,
    )(num0, num1, den0, den1, xr, u, w, bi, wmu_t, bmu, wlv_t, blv, eps)


def _cat_w(p, pad_to=None):
    ws = [p['Wq'].T, p['Wk'].T, p['Wv'].T]
    if pad_to is not None:
        ws = [jnp.pad(wi, ((0, 0), (0, pad_to - wi.shape[1]))) for wi in ws]
    ws.append(p['Ws'].T)
    return jnp.concatenate(ws, axis=1)


def _cat_b(p, pad_to=None):
    bs = [p['bq'], p['bk'], p['bv']]
    if pad_to is not None:
        bs = [jnp.pad(bi, (0, pad_to - bi.shape[0])) for bi in bs]
    bs.append(p['bs'])
    return jnp.concatenate(bs)[None, :]


def _gate_uw(p, c):
    wb = p['Wb'][0]
    return ((wb[:c] + wb[2 * c:])[None, :],
            (wb[c:2 * c] - wb[2 * c:])[None, :])


def _edge(q, k, v, src, dst):
    exh, den = _edge_a(128)(q, k, src, dst)
    num, = _edge_b(128)(v, src, dst, exh)
    return num, den


def kernel(x, edge_attr, edge_index, batch_index, params):
    del edge_attr
    src = edge_index[0]
    dst = edge_index[1]
    convs = params['convs']
    norms = params['norms']

    q, k, v, xr = _proj(x, _cat_w(convs[0]), _cat_b(convs[0]), 128)
    for i in range(3):
        pad = 128 if i == 2 else None
        xw = 64 if i == 2 else 128
        num, den = _edge(q, k, v, src, dst)
        u, w = _gate_uw(convs[i], 128)
        q, k, v, xr = _post_proj(
            num[0], num[1], den[0][:N, None], den[1][:N, None], xr, u, w,
            norms[i]['g'][None, :], norms[i]['b'][None, :],
            _cat_w(convs[i + 1], pad), _cat_b(convs[i + 1], pad),
            128, 128, xw)

    num, den = _edge(q, k, v, src, dst)
    u3, w3 = _gate_uw(convs[3], 64)
    eps = jax.random.normal(jax.random.key(42), (G, 128), jnp.float32)
    return _final(num[0][:, :64], num[1][:, :64],
                  den[0][:N, None], den[1][:N, None], xr,
                  u3, w3, batch_index[:, None].astype(jnp.int32),
                  params['Wmu'].T, params['bmu'][None, :],
                  params['Wlv'].T, params['blv'][None, :], eps)
